# baseline (speedup 1.0000x reference)
import functools

import jax
import jax.numpy as jnp
from jax import lax
from jax.experimental import pallas as pl
from jax.experimental.pallas import tpu as pltpu
from jax.experimental.pallas import tpu_sc as plsc

_NJ, _NG, _C = 50000, 25000, 64
_NJP, _NGP = 50176, 25088
_CH = 512
_NCH_JJ = 1600
_NCH_GJ = 320

_GATES = ("i", "f", "g", "o")
_B = 512
_F32 = jnp.float32

_MESH = dict(core_axis_name="c", subcore_axis_name="s")
_SC_PARAMS = pltpu.CompilerParams(use_tc_tiling_on_sc=False)


def _sc_raw_agg(table, eidx, zrows, n_dst, nch):
    per_core = nch // 2
    per_sub = per_core // 16
    rps = n_dst // 16

    @functools.partial(
        pl.kernel,
        out_type=jax.ShapeDtypeStruct((2, n_dst, 16), _F32),
        mesh=plsc.VectorSubcoreMesh(**_MESH),
        scratch_types=[
            pltpu.VMEM((1, 2, _CH), jnp.int32),
            pltpu.VMEM((_CH, 16), _F32),
            pltpu.VMEM_SHARED((n_dst, 16), _F32),
            pltpu.SemaphoreType.DMA,
        ],
        compiler_params=_SC_PARAMS,
    )
    def k(table_h, eidx_h, z_h, out_h, ebuf, rows, acc, sem):
        cid = lax.axis_index("c")
        sid = lax.axis_index("s")
        r0 = sid * rps
        pltpu.sync_copy(z_h.at[pl.ds(0, rps)], acc.at[pl.ds(r0, rps)])
        plsc.subcore_barrier()
        c0 = cid * per_core + sid * per_sub

        @pl.loop(0, per_sub)
        def _(i):
            pltpu.sync_copy(eidx_h.at[pl.ds(c0 + i, 1)], ebuf)
            pltpu.async_copy(table_h.at[ebuf.at[0, 0]], rows, sem).wait()
            pltpu.sync_copy(rows, acc.at[ebuf.at[0, 1]], add=True)

        plsc.subcore_barrier()

        @pl.when(cid == 0)
        def _():
            pltpu.sync_copy(acc.at[pl.ds(r0, rps)], out_h.at[0].at[pl.ds(r0, rps)])

        @pl.when(cid == 1)
        def _():
            pltpu.sync_copy(acc.at[pl.ds(r0, rps)], out_h.at[1].at[pl.ds(r0, rps)])

    return k(table, eidx, zrows)


def _sc_h_agg(t0, t1, eidx, zrows, n_dst, nch):
    per_sub = nch // 16
    rps = n_dst // 16

    @functools.partial(
        pl.kernel,
        out_type=[jax.ShapeDtypeStruct((n_dst, 32), _F32)] * 2,
        mesh=plsc.VectorSubcoreMesh(**_MESH),
        scratch_types=[
            pltpu.VMEM((1, 2, _CH), jnp.int32),
            pltpu.VMEM((_CH, 32), _F32),
            pltpu.VMEM_SHARED((n_dst, 32), _F32),
            pltpu.SemaphoreType.DMA,
        ],
        compiler_params=_SC_PARAMS,
    )
    def k(t0_h, t1_h, eidx_h, z_h, o0_h, o1_h, ebuf, rows, acc, sem):
        cid = lax.axis_index("c")
        sid = lax.axis_index("s")
        r0 = sid * rps
        pltpu.sync_copy(z_h.at[pl.ds(0, rps)], acc.at[pl.ds(r0, rps)])
        plsc.subcore_barrier()
        c0 = sid * per_sub

        def pipeline(t_h):
            @pl.loop(0, per_sub)
            def _(i):
                pltpu.sync_copy(eidx_h.at[pl.ds(c0 + i, 1)], ebuf)
                pltpu.async_copy(t_h.at[ebuf.at[0, 0]], rows, sem).wait()
                pltpu.sync_copy(rows, acc.at[ebuf.at[0, 1]], add=True)

        @pl.when(cid == 0)
        def _():
            pipeline(t0_h)

        @pl.when(cid == 1)
        def _():
            pipeline(t1_h)

        plsc.subcore_barrier()

        @pl.when(cid == 0)
        def _():
            pltpu.sync_copy(acc.at[pl.ds(r0, rps)], o0_h.at[pl.ds(r0, rps)])

        @pl.when(cid == 1)
        def _():
            pltpu.sync_copy(acc.at[pl.ds(r0, rps)], o1_h.at[pl.ds(r0, rps)])

    return k(t0, t1, eidx, zrows)


def _dot(a, b):
    return lax.dot_general(a, b, (((1,), (0,)), ((), ())),
                           precision=lax.Precision.HIGHEST,
                           preferred_element_type=_F32)


def _tc_cell(n_pad, terms, bias, c_in, out_mode, head=None):
    grid = (n_pad // _B,)
    arrays, specs = [], []

    def add_b(a):
        w = a.shape[1]
        arrays.append(a)
        specs.append(pl.BlockSpec((_B, w), lambda i: (i, 0)))

    def add_f(a):
        arrays.append(a)
        specs.append(pl.BlockSpec(a.shape, lambda i: (0, 0)))

    if c_in is not None:
        add_b(c_in)
    for t in terms:
        if t[0] == "lin":
            for a in t[1]:
                add_b(a)
        elif t[0] == "rawmean":
            add_b(t[1][0]); add_b(t[1][1])
        else:
            add_b(t[1][0]); add_b(t[1][1]); add_b(t[2][0]); add_b(t[2][1])
    for t in terms:
        add_f(t[-1])
    add_f(bias)
    if head is not None:
        add_b(head[0])
        add_f(head[1])

    def body(*refs):
        it = iter(refs)
        nxt = lambda: next(it)[...]
        c_prev = nxt() if c_in is not None else None
        vals = []
        for t in terms:
            if t[0] == "lin":
                vals.append([nxt() for _ in t[1]])
            elif t[0] == "rawmean":
                vals.append([nxt(), nxt()])
            else:
                vals.append([nxt(), nxt(), nxt(), nxt()])
        z = None
        for t, v in zip(terms, vals):
            w = nxt()
            if t[0] == "lin":
                off = 0
                for a in v:
                    part = _dot(a, w[off:off + a.shape[1]])
                    off += a.shape[1]
                    z = part if z is None else z + part
            elif t[0] == "rawmean":
                s = v[0] + v[1]
                r = 1.0 / jnp.maximum(s[:, t[2]:t[2] + 1], 1.0)
                part = _dot(s * r, w)
                z = part if z is None else z + part
            else:
                cnt = v[2] + v[3]
                r = 1.0 / jnp.maximum(cnt[:, t[3]:t[3] + 1], 1.0)
                part = _dot(v[0] * r, w[:32]) + _dot(v[1] * r, w[32:])
                z = part if z is None else z + part
        z = z + nxt()
        ig = jax.nn.sigmoid(z[:, :64])
        fg = jax.nn.sigmoid(z[:, 64:128])
        gg = jnp.tanh(z[:, 128:192])
        og = jax.nn.sigmoid(z[:, 192:256])
        c2 = ig * gg if c_prev is None else fg * c_prev + ig * gg
        h2 = og * jnp.tanh(c2)
        if out_mode == "head":
            xp = nxt()
            wh = nxt()
            zh = _dot(h2, wh[:64]) + _dot(c2, wh[64:128]) + _dot(xp, wh[128:144])
            lane = lax.broadcasted_iota(jnp.int32, zh.shape, 1)
            full = jnp.where(lane < 2, jnp.tanh(zh) / 5.0, jax.nn.sigmoid(zh))
            oref = next(it)
            oref[...] = full[:, :3]
        elif out_mode == "hc":
            h0o, h1o, co = next(it), next(it), next(it)
            h0o[...] = h2[:, :32]
            h1o[...] = h2[:, 32:]
            co[...] = c2
        else:
            h0o, h1o = next(it), next(it)
            h0o[...] = h2[:, :32]
            h1o[...] = h2[:, 32:]

    if out_mode == "head":
        out_shape = jax.ShapeDtypeStruct((n_pad, 3), _F32)
        out_specs = pl.BlockSpec((_B, 3), lambda i: (i, 0))
    elif out_mode == "hc":
        out_shape = [jax.ShapeDtypeStruct((n_pad, 32), _F32)] * 2 + [
            jax.ShapeDtypeStruct((n_pad, 64), _F32)]
        out_specs = [pl.BlockSpec((_B, 32), lambda i: (i, 0))] * 2 + [
            pl.BlockSpec((_B, 64), lambda i: (i, 0))]
    else:
        out_shape = [jax.ShapeDtypeStruct((n_pad, 32), _F32)] * 2
        out_specs = [pl.BlockSpec((_B, 32), lambda i: (i, 0))] * 2

    return pl.pallas_call(body, grid=grid, in_specs=specs,
                          out_specs=out_specs, out_shape=out_shape)(*arrays)


def _prep_edges(ei, nch, dst_pad):
    e = ei.astype(jnp.int32)
    pad = nch * _CH - e.shape[1]
    src = jnp.concatenate([e[0], jnp.zeros((pad,), jnp.int32)])
    dst = jnp.concatenate([e[1], jnp.full((pad,), dst_pad, jnp.int32)])
    return jnp.stack([src.reshape(nch, _CH), dst.reshape(nch, _CH)], axis=1)


def kernel(x_joint, x_grain, params, edge_jj, edge_gj, edge_jg, edge_attr_jj):
    p = params

    def wcat(stem, tail, pad_to=None):
        w = jnp.concatenate([p[f"{stem}_{g}_{tail}"] for g in _GATES], 1)
        if pad_to is not None and w.shape[0] < pad_to:
            w = jnp.zeros((pad_to, w.shape[1]), _F32).at[:w.shape[0]].set(w)
        return w

    def bcat(pre, nt):
        return jnp.concatenate(
            [p[f"{pre}_b_{g}_{nt}"] for g in _GATES]).reshape(1, 256)

    joint_pad = (jnp.zeros((_NJP, 16), _F32)
                 .at[:_NJ, :5].set(x_joint).at[:_NJ, 5].set(1.0))
    grain_pad = (jnp.zeros((_NGP, 16), _F32)
                 .at[:_NG, :8].set(x_grain).at[:_NG, 8].set(1.0))

    e_jj = _prep_edges(edge_jj, _NCH_JJ, _NJP - 1)
    e_gj = _prep_edges(edge_gj, _NCH_GJ, _NJP - 1)
    e_jg = _prep_edges(edge_jg, _NCH_GJ, _NGP - 1)

    z16 = jnp.zeros((_NJP // 16, 16), _F32)
    z32 = jnp.zeros((_NJP // 16, 32), _F32)

    raw_jj = _sc_raw_agg(joint_pad, e_jj, z16, _NJP, _NCH_JJ)
    raw_gj = _sc_raw_agg(grain_pad, e_gj, z16, _NJP, _NCH_GJ)
    raw_jg = _sc_raw_agg(joint_pad, e_jg, z16[:_NGP // 16], _NGP, _NCH_GJ)
    rjj = [raw_jj[0], raw_jj[1]]
    rgj = [raw_gj[0], raw_gj[1]]
    rjg = [raw_jg[0], raw_jg[1]]

    eJ0, eJ1, eJc = _tc_cell(
        _NJP,
        [("lin", [joint_pad], wcat("enc0_Wx", "joint", 16)),
         ("rawmean", rjj, 5, wcat("enc0_We", "jj", 16)),
         ("rawmean", rgj, 8, wcat("enc0_We", "gj", 16))],
        bcat("enc0", "joint"), None, "hc")
    eG0, eG1, eGc = _tc_cell(
        _NGP,
        [("lin", [grain_pad], wcat("enc0_Wx", "grain", 16)),
         ("rawmean", rjg, 5, wcat("enc0_We", "jg", 16))],
        bcat("enc0", "grain"), None, "hc")

    jjE0, jjE1 = _sc_h_agg(eJ0, eJ1, e_jj, z32, _NJP, _NCH_JJ)
    gjE0, gjE1 = _sc_h_agg(eG0, eG1, e_gj, z32, _NJP, _NCH_GJ)

    dJ0, dJ1 = _tc_cell(
        _NJP,
        [("lin", [joint_pad], wcat("dec0_Wx", "joint", 16)),
         ("lin", [eJ0, eJ1], wcat("dec0_Wh", "joint")),
         ("rawmean", rjj, 5, wcat("dec0_We", "jj", 16)),
         ("rawmean", rgj, 8, wcat("dec0_We", "gj", 16))],
        bcat("dec0", "joint"), eJc, "h")
    dG0, dG1 = _tc_cell(
        _NGP,
        [("lin", [grain_pad], wcat("dec0_Wx", "grain", 16)),
         ("lin", [eG0, eG1], wcat("dec0_Wh", "grain")),
         ("rawmean", rjg, 5, wcat("dec0_We", "jg", 16))],
        bcat("dec0", "grain"), eGc, "h")

    e1J0, e1J1, e1Jc = _tc_cell(
        _NJP,
        [("lin", [eJ0, eJ1], wcat("enc1_Wx", "joint")),
         ("hmean", [jjE0, jjE1], rjj, 5, wcat("enc1_We", "jj")),
         ("hmean", [gjE0, gjE1], rgj, 8, wcat("enc1_We", "gj"))],
        bcat("enc1", "joint"), None, "hc")

    jjD0, jjD1 = _sc_h_agg(dJ0, dJ1, e_jj, z32, _NJP, _NCH_JJ)
    gjD0, gjD1 = _sc_h_agg(dG0, dG1, e_gj, z32, _NJP, _NCH_GJ)

    whead = jnp.zeros((144, 128), _F32)
    whead = whead.at[:129, 0:2].set(p["lin1_W"])
    whead = whead.at[:129, 2:3].set(p["lin2_W"])
    whead = whead.at[133, 0:2].set(p["lin1_b"])
    whead = whead.at[133, 2].set(p["lin2_b"][0])

    out = _tc_cell(
        _NJP,
        [("lin", [dJ0, dJ1], wcat("dec1_Wx", "joint")),
         ("lin", [e1J0, e1J1], wcat("dec1_Wh", "joint")),
         ("hmean", [jjD0, jjD1], rjj, 5, wcat("dec1_We", "jj")),
         ("hmean", [gjD0, gjD1], rgj, 8, wcat("dec1_We", "gj"))],
        bcat("dec1", "joint"), e1Jc, "head",
        head=(joint_pad, whead))

    return out[:_NJ]


# speedup vs baseline: 1.0796x; 1.0796x over previous
import functools

import jax
import jax.numpy as jnp
from jax import lax
from jax.experimental import pallas as pl
from jax.experimental.pallas import tpu as pltpu
from jax.experimental.pallas import tpu_sc as plsc

_NJ, _NG, _C = 50000, 25000, 64
_NJP, _NGP = 50176, 25088
_CH = 512
_NCH_JJ = 1600
_NCH_GJ = 320

_GATES = ("i", "f", "g", "o")
_B = 512
_F32 = jnp.float32

_MESH = dict(core_axis_name="c", subcore_axis_name="s")
_SC_PARAMS = pltpu.CompilerParams(use_tc_tiling_on_sc=False)


def _sc_raw_agg(table, eidx, zrows, n_dst, nch):
    per_core = nch // 2
    per_sub = per_core // 16
    rps = n_dst // 16

    @functools.partial(
        pl.kernel,
        out_type=jax.ShapeDtypeStruct((2, n_dst, 16), _F32),
        mesh=plsc.VectorSubcoreMesh(**_MESH),
        scratch_types=[
            pltpu.VMEM((1, 2, _CH), jnp.int32),
            pltpu.VMEM((_CH, 16), _F32),
            pltpu.VMEM_SHARED((n_dst, 16), _F32),
            pltpu.SemaphoreType.DMA,
        ],
        compiler_params=_SC_PARAMS,
    )
    def k(table_h, eidx_h, z_h, out_h, ebuf, rows, acc, sem):
        cid = lax.axis_index("c")
        sid = lax.axis_index("s")
        r0 = sid * rps
        pltpu.sync_copy(z_h.at[pl.ds(0, rps)], acc.at[pl.ds(r0, rps)])
        plsc.subcore_barrier()
        c0 = cid * per_core + sid * per_sub

        @pl.loop(0, per_sub)
        def _(i):
            pltpu.sync_copy(eidx_h.at[pl.ds(c0 + i, 1)], ebuf)
            pltpu.async_copy(table_h.at[ebuf.at[0, 0]], rows, sem).wait()
            pltpu.sync_copy(rows, acc.at[ebuf.at[0, 1]], add=True)

        plsc.subcore_barrier()

        @pl.when(cid == 0)
        def _():
            pltpu.sync_copy(acc.at[pl.ds(r0, rps)], out_h.at[0].at[pl.ds(r0, rps)])

        @pl.when(cid == 1)
        def _():
            pltpu.sync_copy(acc.at[pl.ds(r0, rps)], out_h.at[1].at[pl.ds(r0, rps)])

    return k(table, eidx, zrows)


def _sc_h_comb(tj0, tj1, tg0, tg1, e_jj, e_gj, zrows):
    rps = _NJP // 16
    bf16 = jnp.bfloat16

    @functools.partial(
        pl.kernel,
        out_type=[jax.ShapeDtypeStruct((_NJP, 64), bf16)] * 4,
        mesh=plsc.VectorSubcoreMesh(**_MESH),
        scratch_types=[
            pltpu.VMEM((1, 2, _CH), jnp.int32),
            pltpu.VMEM((_CH, 64), bf16),
            pltpu.VMEM_SHARED((_NJP, 64), bf16),
            pltpu.SemaphoreType.DMA,
        ],
        compiler_params=_SC_PARAMS,
    )
    def k(tj0_h, tj1_h, tg0_h, tg1_h, ejj_h, egj_h, z_h,
          ojj0_h, ojj1_h, ogj0_h, ogj1_h, ebuf, rows, acc, sem):
        cid = lax.axis_index("c")
        sid = lax.axis_index("s")
        r0 = sid * rps

        def phase(t0_h, t1_h, eidx_h, nch, o0_h, o1_h):
            per_sub = nch // 16
            c0 = sid * per_sub
            pltpu.sync_copy(z_h.at[pl.ds(0, rps)], acc.at[pl.ds(r0, rps)])
            plsc.subcore_barrier()

            def body(t_h):
                @pl.loop(0, per_sub)
                def _(i):
                    pltpu.sync_copy(eidx_h.at[pl.ds(c0 + i, 1)], ebuf)
                    pltpu.async_copy(t_h.at[ebuf.at[0, 0]], rows, sem).wait()
                    pltpu.sync_copy(rows, acc.at[ebuf.at[0, 1]], add=True)

            @pl.when(cid == 0)
            def _():
                body(t0_h)

            @pl.when(cid == 1)
            def _():
                body(t1_h)

            plsc.subcore_barrier()

            @pl.when(cid == 0)
            def _():
                pltpu.sync_copy(acc.at[pl.ds(r0, rps)], o0_h.at[pl.ds(r0, rps)])

            @pl.when(cid == 1)
            def _():
                pltpu.sync_copy(acc.at[pl.ds(r0, rps)], o1_h.at[pl.ds(r0, rps)])

        phase(tj0_h, tj1_h, ejj_h, _NCH_JJ, ojj0_h, ojj1_h)
        phase(tg0_h, tg1_h, egj_h, _NCH_GJ, ogj0_h, ogj1_h)

    return k(tj0, tj1, tg0, tg1, e_jj, e_gj, zrows)


def _dot(a, b):
    return lax.dot_general(a, b, (((1,), (0,)), ((), ())),
                           precision=lax.Precision.HIGHEST,
                           preferred_element_type=_F32)


def _tc_cell(n_pad, terms, bias, c_in, out_mode, head=None):
    grid = (n_pad // _B,)
    arrays, specs = [], []

    def add_b(a):
        w = a.shape[1]
        arrays.append(a)
        specs.append(pl.BlockSpec((_B, w), lambda i: (i, 0)))

    def add_f(a):
        arrays.append(a)
        specs.append(pl.BlockSpec(a.shape, lambda i: (0, 0)))

    if c_in is not None:
        add_b(c_in)
    for t in terms:
        if t[0] == "lin":
            for a in t[1]:
                add_b(a)
        elif t[0] == "rawmean":
            add_b(t[1][0]); add_b(t[1][1])
        else:
            add_b(t[1][0]); add_b(t[1][1]); add_b(t[2][0]); add_b(t[2][1])
    for t in terms:
        add_f(t[4] if t[0] == "hmean" else t[-1])
    add_f(bias)
    if head is not None:
        add_b(head[0])
        add_f(head[1])

    def body(*refs):
        it = iter(refs)

        def nxt():
            v = next(it)[...]
            return v.astype(_F32) if v.dtype == jnp.bfloat16 else v
        c_prev = nxt() if c_in is not None else None
        vals = []
        for t in terms:
            if t[0] == "lin":
                vals.append([nxt() for _ in t[1]])
            elif t[0] == "rawmean":
                vals.append([nxt(), nxt()])
            else:
                vals.append([nxt(), nxt(), nxt(), nxt()])
        z = None
        for t, v in zip(terms, vals):
            w = nxt()
            if t[0] == "lin":
                off = 0
                for a in v:
                    part = _dot(a, w[off:off + a.shape[1]])
                    off += a.shape[1]
                    z = part if z is None else z + part
            elif t[0] == "rawmean":
                s = v[0] + v[1]
                r = 1.0 / jnp.maximum(s[:, t[2]:t[2] + 1], 1.0)
                part = _dot(s * r, w)
                z = part if z is None else z + part
            else:
                cnt = v[2] + v[3]
                r = 1.0 / jnp.maximum(cnt[:, t[3]:t[3] + 1], 1.0)
                lo = t[5]
                part = (_dot(v[0][:, lo:lo + 32] * r, w[:32])
                        + _dot(v[1][:, lo:lo + 32] * r, w[32:]))
                z = part if z is None else z + part
        z = z + nxt()
        ig = jax.nn.sigmoid(z[:, :64])
        fg = jax.nn.sigmoid(z[:, 64:128])
        gg = jnp.tanh(z[:, 128:192])
        og = jax.nn.sigmoid(z[:, 192:256])
        c2 = ig * gg if c_prev is None else fg * c_prev + ig * gg
        h2 = og * jnp.tanh(c2)
        if out_mode == "head":
            xp = nxt()
            wh = nxt()
            zh = _dot(h2, wh[:64]) + _dot(c2, wh[64:128]) + _dot(xp, wh[128:144])
            lane = lax.broadcasted_iota(jnp.int32, zh.shape, 1)
            full = jnp.where(lane < 2, jnp.tanh(zh) / 5.0, jax.nn.sigmoid(zh))
            oref = next(it)
            oref[...] = full[:, :3]
        elif out_mode == "hc":
            h0o, h1o, co = next(it), next(it), next(it)
            h0o[...] = h2[:, :32].astype(jnp.bfloat16)
            h1o[...] = h2[:, 32:].astype(jnp.bfloat16)
            co[...] = c2
        else:
            h0o, h1o = next(it), next(it)
            h0o[...] = h2[:, :32].astype(jnp.bfloat16)
            h1o[...] = h2[:, 32:].astype(jnp.bfloat16)

    if out_mode == "head":
        out_shape = jax.ShapeDtypeStruct((n_pad, 3), _F32)
        out_specs = pl.BlockSpec((_B, 3), lambda i: (i, 0))
    elif out_mode == "hc":
        out_shape = [jax.ShapeDtypeStruct((n_pad, 32), jnp.bfloat16)] * 2 + [
            jax.ShapeDtypeStruct((n_pad, 64), _F32)]
        out_specs = [pl.BlockSpec((_B, 32), lambda i: (i, 0))] * 2 + [
            pl.BlockSpec((_B, 64), lambda i: (i, 0))]
    else:
        out_shape = [jax.ShapeDtypeStruct((n_pad, 32), jnp.bfloat16)] * 2
        out_specs = [pl.BlockSpec((_B, 32), lambda i: (i, 0))] * 2

    return pl.pallas_call(body, grid=grid, in_specs=specs,
                          out_specs=out_specs, out_shape=out_shape)(*arrays)


def _prep_edges(ei, nch, dst_pad):
    e = ei.astype(jnp.int32)
    pad = nch * _CH - e.shape[1]
    src = jnp.concatenate([e[0], jnp.zeros((pad,), jnp.int32)])
    dst = jnp.concatenate([e[1], jnp.full((pad,), dst_pad, jnp.int32)])
    return jnp.stack([src.reshape(nch, _CH), dst.reshape(nch, _CH)], axis=1)


def kernel(x_joint, x_grain, params, edge_jj, edge_gj, edge_jg, edge_attr_jj):
    p = params

    def wcat(stem, tail, pad_to=None):
        w = jnp.concatenate([p[f"{stem}_{g}_{tail}"] for g in _GATES], 1)
        if pad_to is not None and w.shape[0] < pad_to:
            w = jnp.zeros((pad_to, w.shape[1]), _F32).at[:w.shape[0]].set(w)
        return w

    def bcat(pre, nt):
        return jnp.concatenate(
            [p[f"{pre}_b_{g}_{nt}"] for g in _GATES]).reshape(1, 256)

    joint_pad = (jnp.zeros((_NJP, 16), _F32)
                 .at[:_NJ, :5].set(x_joint).at[:_NJ, 5].set(1.0))
    grain_pad = (jnp.zeros((_NGP, 16), _F32)
                 .at[:_NG, :8].set(x_grain).at[:_NG, 8].set(1.0))

    e_jj = _prep_edges(edge_jj, _NCH_JJ, _NJP - 1)
    e_gj = _prep_edges(edge_gj, _NCH_GJ, _NJP - 1)
    e_jg = _prep_edges(edge_jg, _NCH_GJ, _NGP - 1)

    z16 = jnp.zeros((_NJP // 16, 16), _F32)
    z64 = jnp.zeros((_NJP // 16, 64), jnp.bfloat16)

    raw_jj = _sc_raw_agg(joint_pad, e_jj, z16, _NJP, _NCH_JJ)
    raw_gj = _sc_raw_agg(grain_pad, e_gj, z16, _NJP, _NCH_GJ)
    raw_jg = _sc_raw_agg(joint_pad, e_jg, z16[:_NGP // 16], _NGP, _NCH_GJ)
    rjj = [raw_jj[0], raw_jj[1]]
    rgj = [raw_gj[0], raw_gj[1]]
    rjg = [raw_jg[0], raw_jg[1]]

    eJ0, eJ1, eJc = _tc_cell(
        _NJP,
        [("lin", [joint_pad], wcat("enc0_Wx", "joint", 16)),
         ("rawmean", rjj, 5, wcat("enc0_We", "jj", 16)),
         ("rawmean", rgj, 8, wcat("enc0_We", "gj", 16))],
        bcat("enc0", "joint"), None, "hc")
    eG0, eG1, eGc = _tc_cell(
        _NGP,
        [("lin", [grain_pad], wcat("enc0_Wx", "grain", 16)),
         ("rawmean", rjg, 5, wcat("enc0_We", "jg", 16))],
        bcat("enc0", "grain"), None, "hc")

    dJ0, dJ1 = _tc_cell(
        _NJP,
        [("lin", [joint_pad], wcat("dec0_Wx", "joint", 16)),
         ("lin", [eJ0, eJ1], wcat("dec0_Wh", "joint")),
         ("rawmean", rjj, 5, wcat("dec0_We", "jj", 16)),
         ("rawmean", rgj, 8, wcat("dec0_We", "gj", 16))],
        bcat("dec0", "joint"), eJc, "h")
    dG0, dG1 = _tc_cell(
        _NGP,
        [("lin", [grain_pad], wcat("dec0_Wx", "grain", 16)),
         ("lin", [eG0, eG1], wcat("dec0_Wh", "grain")),
         ("rawmean", rjg, 5, wcat("dec0_We", "jg", 16))],
        bcat("dec0", "grain"), eGc, "h")

    tj0 = jnp.concatenate([eJ0, dJ0], axis=1)
    tj1 = jnp.concatenate([eJ1, dJ1], axis=1)
    tg0 = jnp.concatenate([eG0, dG0], axis=1)
    tg1 = jnp.concatenate([eG1, dG1], axis=1)
    jjA0, jjA1, gjA0, gjA1 = _sc_h_comb(tj0, tj1, tg0, tg1, e_jj, e_gj, z64)

    e1J0, e1J1, e1Jc = _tc_cell(
        _NJP,
        [("lin", [eJ0, eJ1], wcat("enc1_Wx", "joint")),
         ("hmean", [jjA0, jjA1], rjj, 5, wcat("enc1_We", "jj"), 0),
         ("hmean", [gjA0, gjA1], rgj, 8, wcat("enc1_We", "gj"), 0)],
        bcat("enc1", "joint"), None, "hc")

    whead = jnp.zeros((144, 128), _F32)
    whead = whead.at[:129, 0:2].set(p["lin1_W"])
    whead = whead.at[:129, 2:3].set(p["lin2_W"])
    whead = whead.at[133, 0:2].set(p["lin1_b"])
    whead = whead.at[133, 2].set(p["lin2_b"][0])

    out = _tc_cell(
        _NJP,
        [("lin", [dJ0, dJ1], wcat("dec1_Wx", "joint")),
         ("lin", [e1J0, e1J1], wcat("dec1_Wh", "joint")),
         ("hmean", [jjA0, jjA1], rjj, 5, wcat("dec1_We", "jj"), 32),
         ("hmean", [gjA0, gjA1], rgj, 8, wcat("dec1_We", "gj"), 32)],
        bcat("dec1", "joint"), e1Jc, "head",
        head=(joint_pad, whead))

    return out[:_NJ]
